# baseline (speedup 1.0000x reference)
import jax
import jax.numpy as jnp
from jax.experimental import pallas as pl
from jax.experimental.pallas import tpu as pltpu

_N = 64
_B = 2
_F = 32
_H = 32
_E = _N * (_N - 1)
_C = _N * _B * _H
_G = 4
_S = _N // _G
_GW = 2
_SW = _N // _GW
_F32 = jnp.float32


def _silu(x):
    h = 0.5 * x
    return h * jnp.tanh(h) + h


def _silu_h(h):
    return h * jnp.tanh(h) + h


def _body(x2_ref, wt_ref, wb_ref, b0_ref, w1t_ref, b1_ref, w2t_ref, b2_ref,
          g_ref, bt_ref, em_ref, nid_ref,
          wdn_ref, b0n_ref, w1n_ref, b1n_ref, w2n_ref, b2n_ref, gn_ref,
          btn_ref, nmask_ref,
          msgs_ref, out_ref,
          u_s, v_s, pre_s, stats_s, ss_s, xadj_s):
    s = pl.program_id(0)

    @pl.when(s == 0)
    def _prep():
        u = jnp.dot(x2_ref[...], wt_ref[...], preferred_element_type=_F32)
        v = (jnp.dot(x2_ref[...], wb_ref[...], preferred_element_type=_F32)
             + b0_ref[...])
        u2 = u.reshape(_N * _B, _F, _H)
        v2 = v.reshape(_N * _B, _F, _H)
        u_s[...] = jnp.concatenate([u2[c] for c in range(_N * _B)], axis=1)
        v_s[...] = jnp.stack(
            [jnp.concatenate([v2[2 * n], v2[2 * n + 1]], axis=1)
             for n in range(_N)])
        stats_s[...] = jnp.zeros((_F, 2), _F32)

    @pl.when(s < _S)
    def _edge_compute():
        i0 = _G * s
        vc = jnp.concatenate(
            [jnp.tile(v_s[i0 + t], (1, _N)) for t in range(_G)], axis=1)
        n1 = _silu_h(jnp.tile(u_s[...], (1, _G)) + vc)
        t2 = _silu_h(jnp.dot(w1t_ref[...], n1, preferred_element_type=_F32)
                     + b1_ref[...])
        p = _silu_h(jnp.dot(w2t_ref[...], t2, preferred_element_type=_F32)
                    + b2_ref[...])
        for t in range(_G):
            pre_s[i0 + t] = p[:, t * _C:(t + 1) * _C]
        g = nid_ref[...]
        valid = (g != i0 + 65 * 0)
        for t in range(1, _G):
            valid &= (g != i0 + 65 * t)
        pm = p * valid.astype(_F32)
        stats_s[:, 0:1] += jnp.sum(pm, axis=1, keepdims=True)
        stats_s[:, 1:2] += jnp.sum(pm * p, axis=1, keepdims=True)

    @pl.when(s == _S)
    def _finalize():
        cnt = float(_E * _B * _H)
        mean = stats_s[:, 0:1] / cnt
        var = stats_s[:, 1:2] / cnt - mean * mean
        rstd = jax.lax.rsqrt(var + 1e-5)
        scale = g_ref[...] * rstd
        ss_s[:, 0:1] = scale
        ss_s[:, 1:2] = bt_ref[...] - mean * scale

    @pl.when((s >= _S) & (s < _S + _SW))
    def _edge_write():
        k = s - _S
        sc = ss_s[:, 0:1]
        sh = ss_s[:, 1:2]
        sels = []
        xas = []
        for t in range(_GW):
            y = pre_s[_GW * k + t] * sc + sh
            st = jnp.stack([y[:, 32 * c:32 * (c + 1)]
                            for c in range(_N * _B)])
            st4 = st.reshape(_N, _B, _F, _H)
            jj = jax.lax.broadcasted_iota(jnp.int32, (_N - 1, 1, 1, 1), 0)
            sels.append(jnp.where(jj < _GW * k + t, st4[0:_N - 1], st4[1:_N]))
        em = em_ref[...].reshape(_GW * (_N - 1), _B, 1, 1)
        sel = jnp.concatenate(sels, axis=0) * em
        msgs_ref[...] = sel
        for t in range(_GW):
            xas.append(jnp.sum(sel[t * (_N - 1):(t + 1) * (_N - 1)], axis=0
                               ).reshape(2 * _F, _H) / float(_N - 1))
        xadj_s[pl.ds(k * _GW * 2 * _F, _GW * 2 * _F)] = jnp.concatenate(
            xas, axis=0)

    @pl.when(s == _S + _SW)
    def _node():
        nx = jnp.concatenate([x2_ref[...], xadj_s[...]], axis=1)
        t1 = _silu_h(jnp.dot(nx, wdn_ref[...], preferred_element_type=_F32)
                     + b0n_ref[...])
        t1t = jnp.swapaxes(t1.reshape(_N, _B, _F, _H), -1, -2
                           ).reshape(_N * _B * _H, _F)
        t2 = _silu_h(jnp.dot(t1t, w1n_ref[...], preferred_element_type=_F32)
                     + b1n_ref[...])
        p = _silu_h(jnp.dot(t2, w2n_ref[...], preferred_element_type=_F32)
                    + b2n_ref[...])
        mean = jnp.mean(p, axis=0, keepdims=True)
        var = jnp.mean(p * p, axis=0, keepdims=True) - mean * mean
        ph = (p - mean) * jax.lax.rsqrt(var + 1e-5) * gn_ref[...] + btn_ref[...]
        nm = jnp.transpose(nmask_ref[...])[:, :, None, None]
        out_ref[...] = ph.reshape(_N, _B, _H, _F) * nm


def _full(a):
    return pl.BlockSpec(a.shape, lambda s: (0,) * a.ndim)


def kernel(node_feature, edge_feature, nodes_mask, edges_mask, edge_index,
           edge_params, ne_params):
    del edge_feature, edge_index
    x4 = jnp.transpose(node_feature, (2, 0, 3, 1))
    x2 = x4.reshape(_N * _B * _F, _F)
    wt = 0.5 * edge_params['dim_fc_w'][:_F]
    wb = 0.5 * edge_params['dim_fc_w'][_F:]
    b0 = 0.5 * edge_params['dim_fc_b'].reshape(1, _H)
    w1t = 0.5 * edge_params['fc1_w'].T
    b1 = 0.5 * edge_params['fc1_b'].reshape(2 * _F, 1)
    w2t = 0.5 * edge_params['fc2_w'].T
    b2 = 0.5 * edge_params['fc2_b'].reshape(_F, 1)
    ge = edge_params['bn_gamma'].reshape(_F, 1)
    be = edge_params['bn_beta'].reshape(_F, 1)
    em3 = edges_mask.T.reshape(_SW, _GW * (_N - 1), _B)
    nid = (jnp.arange(_G * _C, dtype=jnp.int32) // (_B * _H))[None, :]

    wdn = 0.5 * ne_params['dim_fc_w']
    b0n = 0.5 * ne_params['dim_fc_b'].reshape(1, _H)
    w1n = 0.5 * ne_params['fc1_w']
    b1n = 0.5 * ne_params['fc1_b'].reshape(1, 2 * _F)
    w2n = 0.5 * ne_params['fc2_w']
    b2n = 0.5 * ne_params['fc2_b'].reshape(1, _F)
    gn = ne_params['bn_gamma'].reshape(1, _F)
    btn = ne_params['bn_beta'].reshape(1, _F)

    ins = (x2, wt, wb, b0, w1t, b1, w2t, b2, ge, be, em3, nid,
           wdn, b0n, w1n, b1n, w2n, b2n, gn, btn, nodes_mask)

    in_specs = [_full(a) for a in ins]
    in_specs[10] = pl.BlockSpec(
        (1, _GW * (_N - 1), _B),
        lambda s: (jnp.clip(s - _S, 0, _SW - 1), 0, 0))

    msgs, out_nbhf = pl.pallas_call(
        _body,
        grid=(_S + _SW + 1,),
        in_specs=in_specs,
        out_specs=[
            pl.BlockSpec((_GW * (_N - 1), _B, _F, _H),
                         lambda s: (jnp.clip(s - _S, 0, _SW - 1), 0, 0, 0)),
            pl.BlockSpec((_N, _B, _H, _F), lambda s: (0, 0, 0, 0)),
        ],
        out_shape=[
            jax.ShapeDtypeStruct((_E, _B, _F, _H), _F32),
            jax.ShapeDtypeStruct((_N, _B, _H, _F), _F32),
        ],
        scratch_shapes=[
            pltpu.VMEM((_F, _C), _F32),
            pltpu.VMEM((_N, _F, _B * _H), _F32),
            pltpu.VMEM((_N, _F, _C), _F32),
            pltpu.VMEM((_F, 2), _F32),
            pltpu.VMEM((_F, 2), _F32),
            pltpu.VMEM((_N * _B * _F, _H), _F32),
        ],
    )(*ins)

    out = jnp.transpose(out_nbhf, (1, 2, 0, 3))
    return out, msgs


# speedup vs baseline: 1.0117x; 1.0117x over previous
import jax
import jax.numpy as jnp
from jax.experimental import pallas as pl
from jax.experimental.pallas import tpu as pltpu

_N = 64
_B = 2
_F = 32
_H = 32
_E = _N * (_N - 1)
_C = _N * _B * _H
_G = 4
_S = _N // _G
_GW = 2
_SW = _N // _GW
_F32 = jnp.float32


def _silu(x):
    h = 0.5 * x
    return h * jnp.tanh(h) + h


def _body(x2_ref, wt_ref, wb_ref, b0_ref, w1t_ref, b1_ref, w2t_ref, b2_ref,
          g_ref, bt_ref, em_ref, nid_ref,
          wdn_ref, b0n_ref, w1n_ref, b1n_ref, w2n_ref, b2n_ref, gn_ref,
          btn_ref, nmask_ref,
          msgs_ref, out_ref,
          u_s, v_s, pre_s, stats_s, ss_s, xadj_s):
    s = pl.program_id(0)

    @pl.when(s == 0)
    def _prep():
        u = jnp.dot(x2_ref[...], wt_ref[...], preferred_element_type=_F32)
        v = (jnp.dot(x2_ref[...], wb_ref[...], preferred_element_type=_F32)
             + b0_ref[...])
        u2 = u.reshape(_N * _B, _F, _H)
        v2 = v.reshape(_N * _B, _F, _H)
        u_s[...] = jnp.concatenate([u2[c] for c in range(_N * _B)], axis=1)
        v_s[...] = jnp.stack(
            [jnp.concatenate([v2[2 * n], v2[2 * n + 1]], axis=1)
             for n in range(_N)])
        stats_s[...] = jnp.zeros((_F, 2), _F32)

    @pl.when(s < _S)
    def _edge_compute():
        i0 = _G * s
        vc = jnp.concatenate(
            [jnp.tile(v_s[i0 + t], (1, _N)) for t in range(_G)], axis=1)
        n1 = _silu(jnp.tile(u_s[...], (1, _G)) + vc)
        t2 = _silu(jnp.dot(w1t_ref[...], n1, preferred_element_type=_F32)
                   + b1_ref[...])
        p = _silu(jnp.dot(w2t_ref[...], t2, preferred_element_type=_F32)
                  + b2_ref[...])
        for t in range(_G):
            pre_s[i0 + t] = p[:, t * _C:(t + 1) * _C]
        g = nid_ref[...]
        valid = (g != i0 + 65 * 0)
        for t in range(1, _G):
            valid &= (g != i0 + 65 * t)
        pm = p * valid.astype(_F32)
        stats_s[:, 0:1] += jnp.sum(pm, axis=1, keepdims=True)
        stats_s[:, 1:2] += jnp.sum(pm * p, axis=1, keepdims=True)

    @pl.when(s == _S)
    def _finalize():
        cnt = float(_E * _B * _H)
        mean = stats_s[:, 0:1] / cnt
        var = stats_s[:, 1:2] / cnt - mean * mean
        rstd = jax.lax.rsqrt(var + 1e-5)
        scale = g_ref[...] * rstd
        ss_s[:, 0:1] = scale
        ss_s[:, 1:2] = bt_ref[...] - mean * scale

    @pl.when((s >= _S) & (s < _S + _SW))
    def _edge_write():
        k = s - _S
        sc = ss_s[:, 0:1]
        sh = ss_s[:, 1:2]
        sels = []
        xas = []
        for t in range(_GW):
            y = pre_s[_GW * k + t] * sc + sh
            st = jnp.stack([y[:, 32 * c:32 * (c + 1)]
                            for c in range(_N * _B)])
            st4 = st.reshape(_N, _B, _F, _H)
            jj = jax.lax.broadcasted_iota(jnp.int32, (_N - 1, 1, 1, 1), 0)
            sels.append(jnp.where(jj < _GW * k + t, st4[0:_N - 1], st4[1:_N]))
        em = em_ref[...].reshape(_GW * (_N - 1), _B, 1, 1)
        sel = jnp.concatenate(sels, axis=0) * em
        msgs_ref[...] = sel
        for t in range(_GW):
            xas.append(jnp.sum(sel[t * (_N - 1):(t + 1) * (_N - 1)], axis=0
                               ).reshape(2 * _F, _H) / float(_N - 1))
        xadj_s[pl.ds(k * _GW * 2 * _F, _GW * 2 * _F)] = jnp.concatenate(
            xas, axis=0)

    @pl.when(s == _S + _SW)
    def _node():
        nx = jnp.concatenate([x2_ref[...], xadj_s[...]], axis=1)
        t1 = _silu(jnp.dot(nx, wdn_ref[...], preferred_element_type=_F32)
                   + b0n_ref[...])
        t1t = jnp.swapaxes(t1.reshape(_N, _B, _F, _H), -1, -2
                           ).reshape(_N * _B * _H, _F)
        t2 = _silu(jnp.dot(t1t, w1n_ref[...], preferred_element_type=_F32)
                   + b1n_ref[...])
        p = _silu(jnp.dot(t2, w2n_ref[...], preferred_element_type=_F32)
                  + b2n_ref[...])
        mean = jnp.mean(p, axis=0, keepdims=True)
        var = jnp.mean(p * p, axis=0, keepdims=True) - mean * mean
        ph = (p - mean) * jax.lax.rsqrt(var + 1e-5) * gn_ref[...] + btn_ref[...]
        nm = jnp.transpose(nmask_ref[...])[:, :, None, None]
        out_ref[...] = ph.reshape(_N, _B, _H, _F) * nm


def _full(a):
    return pl.BlockSpec(a.shape, lambda s: (0,) * a.ndim)


def kernel(node_feature, edge_feature, nodes_mask, edges_mask, edge_index,
           edge_params, ne_params):
    del edge_feature, edge_index
    x4 = jnp.transpose(node_feature, (2, 0, 3, 1))
    x2 = x4.reshape(_N * _B * _F, _F)
    wt = edge_params['dim_fc_w'][:_F]
    wb = edge_params['dim_fc_w'][_F:]
    b0 = edge_params['dim_fc_b'].reshape(1, _H)
    w1t = edge_params['fc1_w'].T
    b1 = edge_params['fc1_b'].reshape(2 * _F, 1)
    w2t = edge_params['fc2_w'].T
    b2 = edge_params['fc2_b'].reshape(_F, 1)
    ge = edge_params['bn_gamma'].reshape(_F, 1)
    be = edge_params['bn_beta'].reshape(_F, 1)
    em3 = edges_mask.T.reshape(_SW, _GW * (_N - 1), _B)
    nid = (jnp.arange(_G * _C, dtype=jnp.int32) // (_B * _H))[None, :]

    wdn = ne_params['dim_fc_w']
    b0n = ne_params['dim_fc_b'].reshape(1, _H)
    w1n = ne_params['fc1_w']
    b1n = ne_params['fc1_b'].reshape(1, 2 * _F)
    w2n = ne_params['fc2_w']
    b2n = ne_params['fc2_b'].reshape(1, _F)
    gn = ne_params['bn_gamma'].reshape(1, _F)
    btn = ne_params['bn_beta'].reshape(1, _F)

    ins = (x2, wt, wb, b0, w1t, b1, w2t, b2, ge, be, em3, nid,
           wdn, b0n, w1n, b1n, w2n, b2n, gn, btn, nodes_mask)

    in_specs = [_full(a) for a in ins]
    in_specs[10] = pl.BlockSpec(
        (1, _GW * (_N - 1), _B),
        lambda s: (jnp.clip(s - _S, 0, _SW - 1), 0, 0))

    msgs, out_nbhf = pl.pallas_call(
        _body,
        grid=(_S + _SW + 1,),
        in_specs=in_specs,
        out_specs=[
            pl.BlockSpec((_GW * (_N - 1), _B, _F, _H),
                         lambda s: (jnp.clip(s - _S, 0, _SW - 1), 0, 0, 0)),
            pl.BlockSpec((_N, _B, _H, _F), lambda s: (0, 0, 0, 0)),
        ],
        out_shape=[
            jax.ShapeDtypeStruct((_E, _B, _F, _H), _F32),
            jax.ShapeDtypeStruct((_N, _B, _H, _F), _F32),
        ],
        scratch_shapes=[
            pltpu.VMEM((_F, _C), _F32),
            pltpu.VMEM((_N, _F, _B * _H), _F32),
            pltpu.VMEM((_N, _F, _C), _F32),
            pltpu.VMEM((_F, 2), _F32),
            pltpu.VMEM((_F, 2), _F32),
            pltpu.VMEM((_N * _B * _F, _H), _F32),
        ],
    )(*ins)

    out = jnp.transpose(out_nbhf, (1, 2, 0, 3))
    return out, msgs
